# baseline (speedup 1.0000x reference)
import jax
import jax.numpy as jnp
from jax.experimental import pallas as pl
from jax.experimental.pallas import tpu as pltpu

_NUM_NODES = 64
_D_FEAT = 256
_D_HID = 1024
_D_OUT = 1024
_BATCH = 4096
_CP = 512
_NP = _D_OUT // _CP
_RB = 512
_NR = _BATCH // _RB


def _body(emb_ref, b1_ref, b2_ref, w1_hbm, w2_hbm, out_hbm,
          w1v, w2v, bcv, sem_w1, sem_w2, sem_out):
    c_w2p0 = pltpu.make_async_copy(
        w2_hbm.at[:, pl.ds(0, _CP)], w2v.at[0], sem_w2.at[0])
    c_w2p0.start()
    c_w1 = pltpu.make_async_copy(w1_hbm, w1v, sem_w1)
    c_w1.start()
    c_w2p1 = pltpu.make_async_copy(
        w2_hbm.at[:, pl.ds(_CP, _CP)], w2v.at[1], sem_w2.at[1])
    c_w2p1.start()

    c_w1.wait()
    h = jnp.dot(emb_ref[...], w1v[...], preferred_element_type=jnp.float32)
    h = jnp.maximum(h + b1_ref[...], 0.0)
    hbar = jnp.mean(h, axis=0, keepdims=True)

    out_copies = []
    w2_copies = (c_w2p0, c_w2p1)
    for p in range(_NP):
        w2_copies[p].wait()
        g = jnp.dot(hbar, w2v[p], preferred_element_type=jnp.float32)
        g = g + b2_ref[:, p * _CP:(p + 1) * _CP]
        bcv[p] = jnp.broadcast_to(g, (_RB, _CP))
        for i in range(_NR):
            c = pltpu.make_async_copy(
                bcv.at[p],
                out_hbm.at[pl.ds(i * _RB, _RB), pl.ds(p * _CP, _CP)],
                sem_out.at[p * _NR + i])
            c.start()
            out_copies.append(c)

    for c in out_copies:
        c.wait()


def kernel(emb, W1, b1, W2, b2, batch_size):
    del batch_size
    b1r = b1.reshape(1, _D_HID)
    b2r = b2.reshape(1, _D_OUT)
    return pl.pallas_call(
        _body,
        in_specs=[
            pl.BlockSpec(memory_space=pltpu.VMEM),
            pl.BlockSpec(memory_space=pltpu.VMEM),
            pl.BlockSpec(memory_space=pltpu.VMEM),
            pl.BlockSpec(memory_space=pl.ANY),
            pl.BlockSpec(memory_space=pl.ANY),
        ],
        out_specs=pl.BlockSpec(memory_space=pl.ANY),
        out_shape=jax.ShapeDtypeStruct((_BATCH, _D_OUT), jnp.float32),
        scratch_shapes=[
            pltpu.VMEM((_D_FEAT, _D_HID), jnp.float32),
            pltpu.VMEM((_NP, _D_HID, _CP), jnp.float32),
            pltpu.VMEM((_NP, _RB, _CP), jnp.float32),
            pltpu.SemaphoreType.DMA,
            pltpu.SemaphoreType.DMA((_NP,)),
            pltpu.SemaphoreType.DMA((_NP * _NR,)),
        ],
    )(emb, b1r, b2r, W1, W2)
